# baseline (speedup 1.0000x reference)
import functools

import jax
import jax.numpy as jnp
from jax import lax
from jax.experimental import pallas as pl
from jax.experimental.pallas import tpu as pltpu
from jax.experimental.pallas import tpu_sc as plsc

_N = 10000
_E = 160000
_NC = 2
_NS = 16
_NW = _NC * _NS
_EPW = _E // _NW
_C = 125
_NCH = _EPW // _C
_RPT = _N // _NS


def _sc_mesh():
    return plsc.VectorSubcoreMesh(
        core_axis_name="c", subcore_axis_name="s", num_cores=_NC, num_subcores=_NS
    )


def _zero_rows(ref, nrows, ncols):
    def body(r, carry):
        for j in range(ncols // 16):
            ref[r, pl.ds(j * 16, 16)] = jnp.zeros((16,), jnp.float32)
        return carry

    lax.fori_loop(0, nrows, body, 0)


def _msgpass(D):
    @functools.partial(
        pl.kernel,
        out_type=jax.ShapeDtypeStruct((_NC, _N, D), jnp.float32),
        mesh=_sc_mesh(),
        scratch_types=[
            pltpu.VMEM((_NCH, _C), jnp.int32),
            pltpu.VMEM((_NCH, _C), jnp.int32),
            pltpu.VMEM((_C, D), jnp.float32),
            pltpu.VMEM((_C, D), jnp.float32),
            pltpu.VMEM_SHARED((_N, D), jnp.float32),
            pltpu.SemaphoreType.DMA,
            pltpu.SemaphoreType.DMA,
            pltpu.SemaphoreType.DMA,
            pltpu.SemaphoreType.DMA,
        ],
    )
    def k(y_hbm, src_hbm, dst_hbm, out_hbm, src_v, dst_v,
          rows0, rows1, acc_sh, gs0, gs1, ss0, ss1):
        cid = lax.axis_index("c")
        sid = lax.axis_index("s")
        wid = cid * _NS + sid

        _zero_rows(rows0, _C, D)
        for b in range(_RPT // _C):
            pltpu.sync_copy(rows0, acc_sh.at[pl.ds(sid * _RPT + b * _C, _C)])
        plsc.subcore_barrier()

        pltpu.sync_copy(src_hbm.at[wid], src_v)
        pltpu.sync_copy(dst_hbm.at[wid], dst_v)

        rows_l = (rows0, rows1)
        gs_l = (gs0, gs1)
        ss_l = (ss0, ss1)
        NB = 2
        for b in range(NB):
            pltpu.async_copy(y_hbm.at[src_v.at[b]], rows_l[b], gs_l[b])

        def step2(i, carry):
            c = i * NB
            not_last = i < (_NCH // NB - 1)
            for b in range(NB):
                pltpu.make_async_copy(y_hbm.at[src_v.at[c + b]], rows_l[b], gs_l[b]).wait()
                pltpu.async_copy(rows_l[b], acc_sh.at[dst_v.at[c + b]], ss_l[b], add=True)

            @pl.when(not_last)
            def _regather():
                for b in range(NB):
                    pltpu.make_async_copy(rows_l[b], acc_sh.at[dst_v.at[c + b]], ss_l[b]).wait()
                    pltpu.async_copy(y_hbm.at[src_v.at[c + NB + b]], rows_l[b], gs_l[b])

            return carry

        lax.fori_loop(0, _NCH // NB, step2, 0)
        for b in range(NB):
            pltpu.make_async_copy(rows_l[b], acc_sh.at[dst_v.at[b]], ss_l[b]).wait()
        plsc.subcore_barrier()

        r0 = pl.multiple_of(sid * 624, 8)
        pltpu.sync_copy(acc_sh.at[pl.ds(r0, 624)], out_hbm.at[cid, pl.ds(r0, 624)])

        @pl.when(sid == _NS - 1)
        def _tail():
            pltpu.sync_copy(acc_sh.at[pl.ds(9984, 16)], out_hbm.at[cid, pl.ds(9984, 16)])

    return k


def _degrees():
    D = 128

    @functools.partial(
        pl.kernel,
        out_type=jax.ShapeDtypeStruct((_NC, _N, D), jnp.float32),
        mesh=_sc_mesh(),
        scratch_types=[
            pltpu.VMEM((_NCH, _C), jnp.int32),
            pltpu.VMEM((_NCH, _C), jnp.int32),
            pltpu.VMEM((_C, D), jnp.float32),
            pltpu.VMEM((_C, D), jnp.float32),
            pltpu.VMEM_SHARED((_N, D), jnp.float32),
        ],
    )
    def k(src_hbm, dst_hbm, out_hbm, src_v, dst_v, e0_v, e1_v, acc_sh):
        cid = lax.axis_index("c")
        sid = lax.axis_index("s")
        wid = cid * _NS + sid

        _zero_rows(e0_v, _C, D)
        for b in range(_RPT // _C):
            pltpu.sync_copy(e0_v, acc_sh.at[pl.ds(sid * _RPT + b * _C, _C)])

        lanes = lax.iota(jnp.int32, 16)

        def fill(r, carry):
            e0_v[r, pl.ds(0, 16)] = jnp.where(lanes == 0, 1.0, 0.0)
            for j in range(1, D // 16):
                e1_v[r, pl.ds(j * 16, 16)] = jnp.zeros((16,), jnp.float32)
            e1_v[r, pl.ds(0, 16)] = jnp.where(lanes == 1, 1.0, 0.0)
            return carry

        lax.fori_loop(0, _C, fill, 0)
        plsc.subcore_barrier()

        pltpu.sync_copy(src_hbm.at[wid], src_v)
        pltpu.sync_copy(dst_hbm.at[wid], dst_v)

        def step(c, carry):
            pltpu.sync_copy(e0_v, acc_sh.at[src_v.at[c]], add=True)
            pltpu.sync_copy(e1_v, acc_sh.at[dst_v.at[c]], add=True)
            return carry

        lax.fori_loop(0, _NCH, step, 0)
        plsc.subcore_barrier()

        r0 = pl.multiple_of(sid * 624, 8)
        pltpu.sync_copy(acc_sh.at[pl.ds(r0, 624)], out_hbm.at[cid, pl.ds(r0, 624)])

        @pl.when(sid == _NS - 1)
        def _tail():
            pltpu.sync_copy(acc_sh.at[pl.ds(9984, 16)], out_hbm.at[cid, pl.ds(9984, 16)])

    return k


_BM = 1000


def _norm_body(s0, s1, d0, d1, ns_o, nd_o):
    ns_o[...] = lax.rsqrt(jnp.maximum(s0[...] + s1[...], 1.0))
    nd_o[...] = lax.rsqrt(jnp.maximum(d0[...] + d1[...], 1.0))


def _normalizers(s0, s1, d0, d1):
    return pl.pallas_call(
        _norm_body,
        out_shape=(
            jax.ShapeDtypeStruct((_N, 1), jnp.float32),
            jax.ShapeDtypeStruct((_N, 1), jnp.float32),
        ),
    )(s0, s1, d0, d1)


def _first_body(x, w, ns, o):
    y = lax.dot_general(x[...], w[...], (((1,), (0,)), ((), ())),
                        preferred_element_type=jnp.float32)
    o[...] = y * ns[...]


def _first_layer(x, w, ns):
    di, do = w.shape
    grid = _N // _BM
    return pl.pallas_call(
        _first_body,
        grid=(grid,),
        in_specs=[
            pl.BlockSpec((_BM, di), lambda i: (i, 0)),
            pl.BlockSpec((di, do), lambda i: (0, 0)),
            pl.BlockSpec((_BM, 1), lambda i: (i, 0)),
        ],
        out_specs=pl.BlockSpec((_BM, do), lambda i: (i, 0)),
        out_shape=jax.ShapeDtypeStruct((_N, do), jnp.float32),
    )(x, w, ns)


def _make_fused_body(pad):
    def _fused_body(p0, p1, nd, b, w, ns, o):
        h = (p0[...] + p1[...]) * nd[...] + b[...]
        h = jnp.maximum(h, 0.0)
        y = lax.dot_general(h, w[...], (((1,), (0,)), ((), ())),
                            preferred_element_type=jnp.float32)
        y = y * ns[...]
        if pad:
            y = jnp.concatenate([y, jnp.zeros((y.shape[0], pad), jnp.float32)], axis=1)
        o[...] = y
    return _fused_body


def _fused_layer(p0, p1, nd, b, w, ns, pad=0):
    di, do = w.shape
    grid = _N // _BM
    return pl.pallas_call(
        _make_fused_body(pad),
        grid=(grid,),
        in_specs=[
            pl.BlockSpec((_BM, di), lambda i: (i, 0)),
            pl.BlockSpec((_BM, di), lambda i: (i, 0)),
            pl.BlockSpec((_BM, 1), lambda i: (i, 0)),
            pl.BlockSpec((1, di), lambda i: (0, 0)),
            pl.BlockSpec((di, do), lambda i: (0, 0)),
            pl.BlockSpec((_BM, 1), lambda i: (i, 0)),
        ],
        out_specs=pl.BlockSpec((_BM, do + pad), lambda i: (i, 0)),
        out_shape=jax.ShapeDtypeStruct((_N, do + pad), jnp.float32),
    )(p0, p1, nd, b, w, ns)


def _final_body(p0, p1, nd, b, o):
    o[...] = (p0[...] + p1[...]) * nd[...] + b[...]


def _final_layer(p0, p1, nd, b):
    d = p0.shape[1]
    grid = _N // _BM
    return pl.pallas_call(
        _final_body,
        grid=(grid,),
        in_specs=[
            pl.BlockSpec((_BM, d), lambda i: (i, 0)),
            pl.BlockSpec((_BM, d), lambda i: (i, 0)),
            pl.BlockSpec((_BM, 1), lambda i: (i, 0)),
            pl.BlockSpec((1, d), lambda i: (0, 0)),
        ],
        out_specs=pl.BlockSpec((_BM, d), lambda i: (i, 0)),
        out_shape=jax.ShapeDtypeStruct((_N, d), jnp.float32),
    )(p0, p1, nd, b)


_BA = 200


def _adj_body(zi, zjt, o):
    t = lax.dot_general(zi[...], zjt[...], (((1,), (0,)), ((), ())),
                        preferred_element_type=jnp.float32)
    o[...] = 1.0 / (1.0 + jnp.exp(-t))


def _adjacency(z):
    d = z.shape[1]
    g = _N // _BA
    return pl.pallas_call(
        _adj_body,
        grid=(g,),
        in_specs=[
            pl.BlockSpec((_BA, d), lambda i: (i, 0)),
            pl.BlockSpec((d, _N), lambda i: (0, 0)),
        ],
        out_specs=pl.BlockSpec((_BA, _N), lambda i: (i, 0)),
        out_shape=jax.ShapeDtypeStruct((_N, _N), jnp.float32),
    )(z, z.T)


def kernel(features, edge_index,
           enc_W0, enc_b0, enc_W1, enc_b1, enc_W2, enc_b2,
           dea_W0, dea_b0, dea_W1, dea_b1, dea_W2, dea_b2,
           des_W0, des_b0, des_W1, des_b1):
    src = edge_index[0].reshape(_NW, _NCH, _C)
    dst = edge_index[1].reshape(_NW, _NCH, _C)

    degp = _degrees()(src, dst)
    nsrc, ndst = _normalizers(degp[0, :, 0:1], degp[1, :, 0:1],
                              degp[0, :, 1:2], degp[1, :, 1:2])

    mp128 = _msgpass(128)

    y = _first_layer(features, enc_W0, nsrc)
    p = mp128(y, src, dst)
    y = _fused_layer(p[0], p[1], ndst, enc_b0.reshape(1, -1), enc_W1, nsrc)
    p = mp128(y, src, dst)
    y = _fused_layer(p[0], p[1], ndst, enc_b1.reshape(1, -1), enc_W2, nsrc, pad=64)
    p = mp128(y, src, dst)
    z = _final_layer(p[0][:, :64], p[1][:, :64], ndst, enc_b2.reshape(1, -1))

    y = _first_layer(z, des_W0, nsrc)
    p = mp128(y, src, dst)
    y = _fused_layer(p[0], p[1], ndst, des_b0.reshape(1, -1), des_W1, nsrc)
    p = mp128(y, src, dst)
    z_ = _final_layer(p[0], p[1], ndst, des_b1.reshape(1, -1))
    adj = _adjacency(z_)

    y = _first_layer(z, dea_W0, nsrc)
    p = mp128(y, src, dst)
    y = _fused_layer(p[0], p[1], ndst, dea_b0.reshape(1, -1), dea_W1, nsrc)
    p = mp128(y, src, dst)
    y = _fused_layer(p[0], p[1], ndst, dea_b1.reshape(1, -1), dea_W2, nsrc)
    p = mp128(y, src, dst)
    recon = _final_layer(p[0], p[1], ndst, dea_b2.reshape(1, -1))

    return z, recon, adj


# speedup vs baseline: 1.1469x; 1.1469x over previous
import functools

import jax
import jax.numpy as jnp
from jax import lax
from jax.experimental import pallas as pl
from jax.experimental.pallas import tpu as pltpu
from jax.experimental.pallas import tpu_sc as plsc

_N = 10000
_E = 160000
_NC = 2
_NS = 16
_NW = _NC * _NS
_EPW = _E // _NW
_C = 125
_NCH = _EPW // _C
_RPT = _N // _NS


def _sc_mesh():
    return plsc.VectorSubcoreMesh(
        core_axis_name="c", subcore_axis_name="s", num_cores=_NC, num_subcores=_NS
    )


def _zero_rows(ref, nrows, ncols):
    def body(r, carry):
        for j in range(ncols // 16):
            ref[r, pl.ds(j * 16, 16)] = jnp.zeros((16,), jnp.float32)
        return carry

    lax.fori_loop(0, nrows, body, 0)


def _msgpass(D):
    @functools.partial(
        pl.kernel,
        out_type=jax.ShapeDtypeStruct((_NC, _N, D), jnp.float32),
        mesh=_sc_mesh(),
        scratch_types=[
            pltpu.VMEM((_NCH, _C), jnp.int32),
            pltpu.VMEM((_NCH, _C), jnp.int32),
            pltpu.VMEM((_C, D), jnp.float32),
            pltpu.VMEM((_C, D), jnp.float32),
            pltpu.VMEM_SHARED((_N, D), jnp.float32),
            pltpu.SemaphoreType.DMA,
            pltpu.SemaphoreType.DMA,
        ],
    )
    def k(y_hbm, src_hbm, dst_hbm, out_hbm, src_v, dst_v,
          rows0, rows1, acc_sh, gs0, gs1):
        cid = lax.axis_index("c")
        sid = lax.axis_index("s")
        wid = cid * _NS + sid

        _zero_rows(rows0, _C, D)
        for b in range(_RPT // _C):
            pltpu.sync_copy(rows0, acc_sh.at[pl.ds(sid * _RPT + b * _C, _C)])
        plsc.subcore_barrier()

        pltpu.sync_copy(src_hbm.at[wid], src_v)
        pltpu.sync_copy(dst_hbm.at[wid], dst_v)

        pltpu.async_copy(y_hbm.at[src_v.at[0]], rows0, gs0)
        pltpu.async_copy(y_hbm.at[src_v.at[1]], rows1, gs1)

        def step2(i, carry):
            c = i * 2
            not_last = i < (_NCH // 2 - 1)
            pltpu.make_async_copy(y_hbm.at[src_v.at[c]], rows0, gs0).wait()
            pltpu.sync_copy(rows0, acc_sh.at[dst_v.at[c]], add=True)

            @pl.when(not_last)
            def _g0():
                pltpu.async_copy(y_hbm.at[src_v.at[c + 2]], rows0, gs0)

            pltpu.make_async_copy(y_hbm.at[src_v.at[c + 1]], rows1, gs1).wait()
            pltpu.sync_copy(rows1, acc_sh.at[dst_v.at[c + 1]], add=True)

            @pl.when(not_last)
            def _g1():
                pltpu.async_copy(y_hbm.at[src_v.at[c + 3]], rows1, gs1)

            return carry

        lax.fori_loop(0, _NCH // 2, step2, 0)
        plsc.subcore_barrier()

        r0 = pl.multiple_of(sid * 624, 8)
        pltpu.sync_copy(acc_sh.at[pl.ds(r0, 624)], out_hbm.at[cid, pl.ds(r0, 624)])

        @pl.when(sid == _NS - 1)
        def _tail():
            pltpu.sync_copy(acc_sh.at[pl.ds(9984, 16)], out_hbm.at[cid, pl.ds(9984, 16)])

    return k


def _degrees():
    D = 128

    @functools.partial(
        pl.kernel,
        out_type=jax.ShapeDtypeStruct((_NC, _N, D), jnp.float32),
        mesh=_sc_mesh(),
        scratch_types=[
            pltpu.VMEM((_NCH, _C), jnp.int32),
            pltpu.VMEM((_NCH, _C), jnp.int32),
            pltpu.VMEM((_C, D), jnp.float32),
            pltpu.VMEM((_C, D), jnp.float32),
            pltpu.VMEM_SHARED((_N, D), jnp.float32),
            pltpu.SemaphoreType.DMA,
        ],
    )
    def k(src_hbm, dst_hbm, out_hbm, src_v, dst_v, e0_v, e1_v, acc_sh, ss):
        cid = lax.axis_index("c")
        sid = lax.axis_index("s")
        wid = cid * _NS + sid

        _zero_rows(e0_v, _C, D)
        for b in range(_RPT // _C):
            pltpu.sync_copy(e0_v, acc_sh.at[pl.ds(sid * _RPT + b * _C, _C)])

        lanes = lax.iota(jnp.int32, 16)

        def fill(r, carry):
            e0_v[r, pl.ds(0, 16)] = jnp.where(lanes == 0, 1.0, 0.0)
            for j in range(1, D // 16):
                e1_v[r, pl.ds(j * 16, 16)] = jnp.zeros((16,), jnp.float32)
            e1_v[r, pl.ds(0, 16)] = jnp.where(lanes == 1, 1.0, 0.0)
            return carry

        lax.fori_loop(0, _C, fill, 0)
        plsc.subcore_barrier()

        pltpu.sync_copy(src_hbm.at[wid], src_v)
        pltpu.sync_copy(dst_hbm.at[wid], dst_v)

        def step(c, carry):
            pltpu.async_copy(e0_v, acc_sh.at[src_v.at[c]], ss, add=True)
            pltpu.sync_copy(e1_v, acc_sh.at[dst_v.at[c]], add=True)
            pltpu.make_async_copy(e0_v, acc_sh.at[src_v.at[c]], ss).wait()
            return carry

        lax.fori_loop(0, _NCH, step, 0)
        plsc.subcore_barrier()

        r0 = pl.multiple_of(sid * 624, 8)
        pltpu.sync_copy(acc_sh.at[pl.ds(r0, 624)], out_hbm.at[cid, pl.ds(r0, 624)])

        @pl.when(sid == _NS - 1)
        def _tail():
            pltpu.sync_copy(acc_sh.at[pl.ds(9984, 16)], out_hbm.at[cid, pl.ds(9984, 16)])

    return k


_BM = 1000


def _norm_body(s0, s1, d0, d1, ns_o, nd_o):
    ns_o[...] = lax.rsqrt(jnp.maximum(s0[...] + s1[...], 1.0))
    nd_o[...] = lax.rsqrt(jnp.maximum(d0[...] + d1[...], 1.0))


def _normalizers(s0, s1, d0, d1):
    return pl.pallas_call(
        _norm_body,
        out_shape=(
            jax.ShapeDtypeStruct((_N, 1), jnp.float32),
            jax.ShapeDtypeStruct((_N, 1), jnp.float32),
        ),
    )(s0, s1, d0, d1)


def _first_body(x, w, ns, o):
    y = lax.dot_general(x[...], w[...], (((1,), (0,)), ((), ())),
                        preferred_element_type=jnp.float32)
    o[...] = y * ns[...]


def _first_layer(x, w, ns):
    di, do = w.shape
    grid = _N // _BM
    return pl.pallas_call(
        _first_body,
        grid=(grid,),
        in_specs=[
            pl.BlockSpec((_BM, di), lambda i: (i, 0)),
            pl.BlockSpec((di, do), lambda i: (0, 0)),
            pl.BlockSpec((_BM, 1), lambda i: (i, 0)),
        ],
        out_specs=pl.BlockSpec((_BM, do), lambda i: (i, 0)),
        out_shape=jax.ShapeDtypeStruct((_N, do), jnp.float32),
    )(x, w, ns)


def _make_fused_body(pad):
    def _fused_body(p0, p1, nd, b, w, ns, o):
        h = (p0[...] + p1[...]) * nd[...] + b[...]
        h = jnp.maximum(h, 0.0)
        y = lax.dot_general(h, w[...], (((1,), (0,)), ((), ())),
                            preferred_element_type=jnp.float32)
        y = y * ns[...]
        if pad:
            y = jnp.concatenate([y, jnp.zeros((y.shape[0], pad), jnp.float32)], axis=1)
        o[...] = y
    return _fused_body


def _fused_layer(p0, p1, nd, b, w, ns, pad=0):
    di, do = w.shape
    grid = _N // _BM
    return pl.pallas_call(
        _make_fused_body(pad),
        grid=(grid,),
        in_specs=[
            pl.BlockSpec((_BM, di), lambda i: (i, 0)),
            pl.BlockSpec((_BM, di), lambda i: (i, 0)),
            pl.BlockSpec((_BM, 1), lambda i: (i, 0)),
            pl.BlockSpec((1, di), lambda i: (0, 0)),
            pl.BlockSpec((di, do), lambda i: (0, 0)),
            pl.BlockSpec((_BM, 1), lambda i: (i, 0)),
        ],
        out_specs=pl.BlockSpec((_BM, do + pad), lambda i: (i, 0)),
        out_shape=jax.ShapeDtypeStruct((_N, do + pad), jnp.float32),
    )(p0, p1, nd, b, w, ns)


def _final_body(p0, p1, nd, b, o):
    o[...] = (p0[...] + p1[...]) * nd[...] + b[...]


def _final_layer(p0, p1, nd, b):
    d = p0.shape[1]
    grid = _N // _BM
    return pl.pallas_call(
        _final_body,
        grid=(grid,),
        in_specs=[
            pl.BlockSpec((_BM, d), lambda i: (i, 0)),
            pl.BlockSpec((_BM, d), lambda i: (i, 0)),
            pl.BlockSpec((_BM, 1), lambda i: (i, 0)),
            pl.BlockSpec((1, d), lambda i: (0, 0)),
        ],
        out_specs=pl.BlockSpec((_BM, d), lambda i: (i, 0)),
        out_shape=jax.ShapeDtypeStruct((_N, d), jnp.float32),
    )(p0, p1, nd, b)


_BA = 200


def _adj_body(zi, zjt, o):
    t = lax.dot_general(zi[...], zjt[...], (((1,), (0,)), ((), ())),
                        preferred_element_type=jnp.float32)
    o[...] = 1.0 / (1.0 + jnp.exp(-t))


def _adjacency(z):
    d = z.shape[1]
    g = _N // _BA
    return pl.pallas_call(
        _adj_body,
        grid=(g,),
        in_specs=[
            pl.BlockSpec((_BA, d), lambda i: (i, 0)),
            pl.BlockSpec((d, _N), lambda i: (0, 0)),
        ],
        out_specs=pl.BlockSpec((_BA, _N), lambda i: (i, 0)),
        out_shape=jax.ShapeDtypeStruct((_N, _N), jnp.float32),
    )(z, z.T)


def kernel(features, edge_index,
           enc_W0, enc_b0, enc_W1, enc_b1, enc_W2, enc_b2,
           dea_W0, dea_b0, dea_W1, dea_b1, dea_W2, dea_b2,
           des_W0, des_b0, des_W1, des_b1):
    src = edge_index[0].reshape(_NW, _NCH, _C)
    dst = edge_index[1].reshape(_NW, _NCH, _C)

    degp = _degrees()(src, dst)
    nsrc, ndst = _normalizers(degp[0, :, 0:1], degp[1, :, 0:1],
                              degp[0, :, 1:2], degp[1, :, 1:2])

    mp128 = _msgpass(128)

    y = _first_layer(features, enc_W0, nsrc)
    p = mp128(y, src, dst)
    y = _fused_layer(p[0], p[1], ndst, enc_b0.reshape(1, -1), enc_W1, nsrc)
    p = mp128(y, src, dst)
    y = _fused_layer(p[0], p[1], ndst, enc_b1.reshape(1, -1), enc_W2, nsrc, pad=64)
    p = mp128(y, src, dst)
    z = _final_layer(p[0][:, :64], p[1][:, :64], ndst, enc_b2.reshape(1, -1))

    y = _first_layer(z, des_W0, nsrc)
    p = mp128(y, src, dst)
    y = _fused_layer(p[0], p[1], ndst, des_b0.reshape(1, -1), des_W1, nsrc)
    p = mp128(y, src, dst)
    z_ = _final_layer(p[0], p[1], ndst, des_b1.reshape(1, -1))
    adj = _adjacency(z_)

    y = _first_layer(z, dea_W0, nsrc)
    p = mp128(y, src, dst)
    y = _fused_layer(p[0], p[1], ndst, dea_b0.reshape(1, -1), dea_W1, nsrc)
    p = mp128(y, src, dst)
    y = _fused_layer(p[0], p[1], ndst, dea_b1.reshape(1, -1), dea_W2, nsrc)
    p = mp128(y, src, dst)
    recon = _final_layer(p[0], p[1], ndst, dea_b2.reshape(1, -1))

    return z, recon, adj


# speedup vs baseline: 1.1789x; 1.0279x over previous
import functools

import jax
import jax.numpy as jnp
from jax import lax
from jax.experimental import pallas as pl
from jax.experimental.pallas import tpu as pltpu
from jax.experimental.pallas import tpu_sc as plsc

_N = 10000
_E = 160000
_NC = 2
_NS = 16
_NW = _NC * _NS
_EPW = _E // _NW
_C = 125
_NCH = _EPW // _C
_RPT = _N // _NS


def _sc_mesh():
    return plsc.VectorSubcoreMesh(
        core_axis_name="c", subcore_axis_name="s", num_cores=_NC, num_subcores=_NS
    )


def _zero_rows(ref, nrows, ncols):
    def body(r, carry):
        for j in range(ncols // 16):
            ref[r, pl.ds(j * 16, 16)] = jnp.zeros((16,), jnp.float32)
        return carry

    lax.fori_loop(0, nrows, body, 0)


def _msgpass(D):
    @functools.partial(
        pl.kernel,
        out_type=jax.ShapeDtypeStruct((_NC, _N, D), jnp.float32),
        mesh=_sc_mesh(),
        scratch_types=[
            pltpu.VMEM((_NCH, _C), jnp.int32),
            pltpu.VMEM((_NCH, _C), jnp.int32),
            pltpu.VMEM((_C, D), jnp.float32),
            pltpu.VMEM((_C, D), jnp.float32),
            pltpu.VMEM_SHARED((_N, D), jnp.float32),
            pltpu.SemaphoreType.DMA,
            pltpu.SemaphoreType.DMA,
        ],
    )
    def k(y_hbm, src_hbm, dst_hbm, out_hbm, src_v, dst_v,
          rows0, rows1, acc_sh, gs0, gs1):
        cid = lax.axis_index("c")
        sid = lax.axis_index("s")
        wid = cid * _NS + sid

        _zero_rows(rows0, _C, D)
        for b in range(_RPT // _C):
            pltpu.sync_copy(rows0, acc_sh.at[pl.ds(sid * _RPT + b * _C, _C)])
        plsc.subcore_barrier()

        pltpu.sync_copy(src_hbm.at[wid], src_v)
        pltpu.sync_copy(dst_hbm.at[wid], dst_v)

        pltpu.async_copy(y_hbm.at[src_v.at[0]], rows0, gs0)
        pltpu.async_copy(y_hbm.at[src_v.at[1]], rows1, gs1)

        def step2(i, carry):
            c = i * 2
            not_last = i < (_NCH // 2 - 1)
            pltpu.make_async_copy(y_hbm.at[src_v.at[c]], rows0, gs0).wait()
            pltpu.sync_copy(rows0, acc_sh.at[dst_v.at[c]], add=True)

            @pl.when(not_last)
            def _g0():
                pltpu.async_copy(y_hbm.at[src_v.at[c + 2]], rows0, gs0)

            pltpu.make_async_copy(y_hbm.at[src_v.at[c + 1]], rows1, gs1).wait()
            pltpu.sync_copy(rows1, acc_sh.at[dst_v.at[c + 1]], add=True)

            @pl.when(not_last)
            def _g1():
                pltpu.async_copy(y_hbm.at[src_v.at[c + 3]], rows1, gs1)

            return carry

        lax.fori_loop(0, _NCH // 2, step2, 0)
        plsc.subcore_barrier()

        r0 = pl.multiple_of(sid * 624, 8)
        pltpu.sync_copy(acc_sh.at[pl.ds(r0, 624)], out_hbm.at[cid, pl.ds(r0, 624)])

        @pl.when(sid == _NS - 1)
        def _tail():
            pltpu.sync_copy(acc_sh.at[pl.ds(9984, 16)], out_hbm.at[cid, pl.ds(9984, 16)])

    return k


def _degrees():
    D = 128

    @functools.partial(
        pl.kernel,
        out_type=jax.ShapeDtypeStruct((_NC, _N, D), jnp.float32),
        mesh=_sc_mesh(),
        scratch_types=[
            pltpu.VMEM((_NCH, _C), jnp.int32),
            pltpu.VMEM((_NCH, _C), jnp.int32),
            pltpu.VMEM((_C, D), jnp.float32),
            pltpu.VMEM((_C, D), jnp.float32),
            pltpu.VMEM_SHARED((_N, D), jnp.float32),
            pltpu.SemaphoreType.DMA,
        ],
    )
    def k(src_hbm, dst_hbm, out_hbm, src_v, dst_v, e0_v, e1_v, acc_sh, ss):
        cid = lax.axis_index("c")
        sid = lax.axis_index("s")
        wid = cid * _NS + sid

        _zero_rows(e0_v, _C, D)
        for b in range(_RPT // _C):
            pltpu.sync_copy(e0_v, acc_sh.at[pl.ds(sid * _RPT + b * _C, _C)])

        lanes = lax.iota(jnp.int32, 16)

        def fill(r, carry):
            e0_v[r, pl.ds(0, 16)] = jnp.where(lanes == 0, 1.0, 0.0)
            for j in range(1, D // 16):
                e1_v[r, pl.ds(j * 16, 16)] = jnp.zeros((16,), jnp.float32)
            e1_v[r, pl.ds(0, 16)] = jnp.where(lanes == 1, 1.0, 0.0)
            return carry

        lax.fori_loop(0, _C, fill, 0)
        plsc.subcore_barrier()

        pltpu.sync_copy(src_hbm.at[wid], src_v)
        pltpu.sync_copy(dst_hbm.at[wid], dst_v)

        def step(c, carry):
            pltpu.async_copy(e0_v, acc_sh.at[src_v.at[c]], ss, add=True)
            pltpu.sync_copy(e1_v, acc_sh.at[dst_v.at[c]], add=True)
            pltpu.make_async_copy(e0_v, acc_sh.at[src_v.at[c]], ss).wait()
            return carry

        lax.fori_loop(0, _NCH, step, 0)
        plsc.subcore_barrier()

        r0 = pl.multiple_of(sid * 624, 8)
        pltpu.sync_copy(acc_sh.at[pl.ds(r0, 624)], out_hbm.at[cid, pl.ds(r0, 624)])

        @pl.when(sid == _NS - 1)
        def _tail():
            pltpu.sync_copy(acc_sh.at[pl.ds(9984, 16)], out_hbm.at[cid, pl.ds(9984, 16)])

    return k


_BM = 1000


def _norm_body(s0, s1, d0, d1, ns_o, nd_o):
    ns_o[...] = lax.rsqrt(jnp.maximum(s0[...] + s1[...], 1.0))
    nd_o[...] = lax.rsqrt(jnp.maximum(d0[...] + d1[...], 1.0))


def _normalizers(s0, s1, d0, d1):
    return pl.pallas_call(
        _norm_body,
        out_shape=(
            jax.ShapeDtypeStruct((_N, 1), jnp.float32),
            jax.ShapeDtypeStruct((_N, 1), jnp.float32),
        ),
    )(s0, s1, d0, d1)


def _first_body(x, w, ns, o):
    y = lax.dot_general(x[...], w[...], (((1,), (0,)), ((), ())),
                        preferred_element_type=jnp.float32)
    o[...] = y * ns[...]


def _first_layer(x, w, ns):
    di, do = w.shape
    grid = _N // _BM
    return pl.pallas_call(
        _first_body,
        grid=(grid,),
        in_specs=[
            pl.BlockSpec((_BM, di), lambda i: (i, 0)),
            pl.BlockSpec((di, do), lambda i: (0, 0)),
            pl.BlockSpec((_BM, 1), lambda i: (i, 0)),
        ],
        out_specs=pl.BlockSpec((_BM, do), lambda i: (i, 0)),
        out_shape=jax.ShapeDtypeStruct((_N, do), jnp.float32),
    )(x, w, ns)


def _make_fused_body(pad):
    def _fused_body(p0, p1, nd, b, w, ns, o):
        h = (p0[...] + p1[...]) * nd[...] + b[...]
        h = jnp.maximum(h, 0.0)
        y = lax.dot_general(h, w[...], (((1,), (0,)), ((), ())),
                            preferred_element_type=jnp.float32)
        y = y * ns[...]
        if pad:
            y = jnp.concatenate([y, jnp.zeros((y.shape[0], pad), jnp.float32)], axis=1)
        o[...] = y
    return _fused_body


def _fused_layer(p0, p1, nd, b, w, ns, pad=0):
    di, do = w.shape
    grid = _N // _BM
    return pl.pallas_call(
        _make_fused_body(pad),
        grid=(grid,),
        in_specs=[
            pl.BlockSpec((_BM, di), lambda i: (i, 0)),
            pl.BlockSpec((_BM, di), lambda i: (i, 0)),
            pl.BlockSpec((_BM, 1), lambda i: (i, 0)),
            pl.BlockSpec((1, di), lambda i: (0, 0)),
            pl.BlockSpec((di, do), lambda i: (0, 0)),
            pl.BlockSpec((_BM, 1), lambda i: (i, 0)),
        ],
        out_specs=pl.BlockSpec((_BM, do + pad), lambda i: (i, 0)),
        out_shape=jax.ShapeDtypeStruct((_N, do + pad), jnp.float32),
    )(p0, p1, nd, b, w, ns)


def _zbranch_body(p0, p1, nd, b, wa, ws, ns, oz, oa, os_):
    zz = (p0[...] + p1[...]) * nd[...] + b[...]
    oz[...] = zz
    oa[...] = lax.dot_general(zz, wa[...], (((1,), (0,)), ((), ())),
                              preferred_element_type=jnp.float32) * ns[...]
    os_[...] = lax.dot_general(zz, ws[...], (((1,), (0,)), ((), ())),
                               preferred_element_type=jnp.float32) * ns[...]


def _zbranch(p0, p1, nd, b, wa, ws, ns):
    di, do = wa.shape
    grid = _N // _BM
    return pl.pallas_call(
        _zbranch_body,
        grid=(grid,),
        in_specs=[
            pl.BlockSpec((_BM, di), lambda i: (i, 0)),
            pl.BlockSpec((_BM, di), lambda i: (i, 0)),
            pl.BlockSpec((_BM, 1), lambda i: (i, 0)),
            pl.BlockSpec((1, di), lambda i: (0, 0)),
            pl.BlockSpec((di, do), lambda i: (0, 0)),
            pl.BlockSpec((di, do), lambda i: (0, 0)),
            pl.BlockSpec((_BM, 1), lambda i: (i, 0)),
        ],
        out_specs=[
            pl.BlockSpec((_BM, di), lambda i: (i, 0)),
            pl.BlockSpec((_BM, do), lambda i: (i, 0)),
            pl.BlockSpec((_BM, do), lambda i: (i, 0)),
        ],
        out_shape=[
            jax.ShapeDtypeStruct((_N, di), jnp.float32),
            jax.ShapeDtypeStruct((_N, do), jnp.float32),
            jax.ShapeDtypeStruct((_N, do), jnp.float32),
        ],
    )(p0, p1, nd, b, wa, ws, ns)


def _final_body(p0, p1, nd, b, o):
    o[...] = (p0[...] + p1[...]) * nd[...] + b[...]


def _final_layer(p0, p1, nd, b):
    d = p0.shape[1]
    grid = _N // _BM
    return pl.pallas_call(
        _final_body,
        grid=(grid,),
        in_specs=[
            pl.BlockSpec((_BM, d), lambda i: (i, 0)),
            pl.BlockSpec((_BM, d), lambda i: (i, 0)),
            pl.BlockSpec((_BM, 1), lambda i: (i, 0)),
            pl.BlockSpec((1, d), lambda i: (0, 0)),
        ],
        out_specs=pl.BlockSpec((_BM, d), lambda i: (i, 0)),
        out_shape=jax.ShapeDtypeStruct((_N, d), jnp.float32),
    )(p0, p1, nd, b)


_BA = 400


def _adj_body(zi, zjt, o):
    t = lax.dot_general(zi[...], zjt[...], (((1,), (0,)), ((), ())),
                        preferred_element_type=jnp.float32)
    o[...] = 1.0 / (1.0 + jnp.exp(-t))


def _adjacency(z):
    d = z.shape[1]
    g = _N // _BA
    return pl.pallas_call(
        _adj_body,
        grid=(g,),
        in_specs=[
            pl.BlockSpec((_BA, d), lambda i: (i, 0)),
            pl.BlockSpec((d, _N), lambda i: (0, 0)),
        ],
        out_specs=pl.BlockSpec((_BA, _N), lambda i: (i, 0)),
        out_shape=jax.ShapeDtypeStruct((_N, _N), jnp.float32),
    )(z, z.T)


def kernel(features, edge_index,
           enc_W0, enc_b0, enc_W1, enc_b1, enc_W2, enc_b2,
           dea_W0, dea_b0, dea_W1, dea_b1, dea_W2, dea_b2,
           des_W0, des_b0, des_W1, des_b1):
    src = edge_index[0].reshape(_NW, _NCH, _C)
    dst = edge_index[1].reshape(_NW, _NCH, _C)

    degp = _degrees()(src, dst)
    nsrc, ndst = _normalizers(degp[0, :, 0:1], degp[1, :, 0:1],
                              degp[0, :, 1:2], degp[1, :, 1:2])

    mp128 = _msgpass(128)

    y = _first_layer(features, enc_W0, nsrc)
    p = mp128(y, src, dst)
    y = _fused_layer(p[0], p[1], ndst, enc_b0.reshape(1, -1), enc_W1, nsrc)
    p = mp128(y, src, dst)
    y = _fused_layer(p[0], p[1], ndst, enc_b1.reshape(1, -1), enc_W2, nsrc, pad=64)
    p = mp128(y, src, dst)
    z, y_dea, y_des = _zbranch(p[0][:, :64], p[1][:, :64], ndst,
                               enc_b2.reshape(1, -1), dea_W0, des_W0, nsrc)

    p = mp128(y_des, src, dst)
    y = _fused_layer(p[0], p[1], ndst, des_b0.reshape(1, -1), des_W1, nsrc)
    p = mp128(y, src, dst)
    z_ = _final_layer(p[0], p[1], ndst, des_b1.reshape(1, -1))
    adj = _adjacency(z_)

    p = mp128(y_dea, src, dst)
    y = _fused_layer(p[0], p[1], ndst, dea_b0.reshape(1, -1), dea_W1, nsrc)
    p = mp128(y, src, dst)
    y = _fused_layer(p[0], p[1], ndst, dea_b1.reshape(1, -1), dea_W2, nsrc)
    p = mp128(y, src, dst)
    recon = _final_layer(p[0], p[1], ndst, dea_b2.reshape(1, -1))

    return z, recon, adj


# speedup vs baseline: 1.1965x; 1.0149x over previous
import functools

import jax
import jax.numpy as jnp
from jax import lax
from jax.experimental import pallas as pl
from jax.experimental.pallas import tpu as pltpu
from jax.experimental.pallas import tpu_sc as plsc

_N = 10000
_E = 160000
_NC = 2
_NS = 16
_NW = _NC * _NS
_EPW = _E // _NW
_C = 125
_NCH = _EPW // _C
_RPT = _N // _NS


def _sc_mesh():
    return plsc.VectorSubcoreMesh(
        core_axis_name="c", subcore_axis_name="s", num_cores=_NC, num_subcores=_NS
    )


def _zero_rows(ref, nrows, ncols):
    def body(r, carry):
        for j in range(ncols // 16):
            ref[r, pl.ds(j * 16, 16)] = jnp.zeros((16,), jnp.float32)
        return carry

    lax.fori_loop(0, nrows, body, 0)


def _msgpass(D):
    @functools.partial(
        pl.kernel,
        out_type=jax.ShapeDtypeStruct((_NC, _N, D), jnp.float32),
        mesh=_sc_mesh(),
        scratch_types=[
            pltpu.VMEM((_NCH, _C), jnp.int32),
            pltpu.VMEM((_NCH, _C), jnp.int32),
            pltpu.VMEM((_C, D), jnp.float32),
            pltpu.VMEM((_C, D), jnp.float32),
            pltpu.VMEM_SHARED((_N, D), jnp.float32),
            pltpu.SemaphoreType.DMA,
            pltpu.SemaphoreType.DMA,
            pltpu.SemaphoreType.DMA,
        ],
    )
    def k(y_hbm, src_hbm, dst_hbm, out_hbm, src_v, dst_v,
          rows0, rows1, acc_sh, gs0, gs1, zs):
        cid = lax.axis_index("c")
        sid = lax.axis_index("s")
        wid = cid * _NS + sid

        pltpu.async_copy(src_hbm.at[wid], src_v, gs0)
        pltpu.async_copy(dst_hbm.at[wid], dst_v, gs1)
        _zero_rows(rows0, _C, D)
        for b in range(_RPT // _C):
            pltpu.async_copy(rows0, acc_sh.at[pl.ds(sid * _RPT + b * _C, _C)], zs)
        for b in range(_RPT // _C):
            pltpu.make_async_copy(rows0, acc_sh.at[pl.ds(sid * _RPT + b * _C, _C)], zs).wait()
        pltpu.make_async_copy(src_hbm.at[wid], src_v, gs0).wait()
        pltpu.make_async_copy(dst_hbm.at[wid], dst_v, gs1).wait()
        plsc.subcore_barrier()

        pltpu.async_copy(y_hbm.at[src_v.at[0]], rows0, gs0)
        pltpu.async_copy(y_hbm.at[src_v.at[1]], rows1, gs1)

        def step2(i, carry):
            c = i * 2
            not_last = i < (_NCH // 2 - 1)
            pltpu.make_async_copy(y_hbm.at[src_v.at[c]], rows0, gs0).wait()
            pltpu.sync_copy(rows0, acc_sh.at[dst_v.at[c]], add=True)

            @pl.when(not_last)
            def _g0():
                pltpu.async_copy(y_hbm.at[src_v.at[c + 2]], rows0, gs0)

            pltpu.make_async_copy(y_hbm.at[src_v.at[c + 1]], rows1, gs1).wait()
            pltpu.sync_copy(rows1, acc_sh.at[dst_v.at[c + 1]], add=True)

            @pl.when(not_last)
            def _g1():
                pltpu.async_copy(y_hbm.at[src_v.at[c + 3]], rows1, gs1)

            return carry

        lax.fori_loop(0, _NCH // 2, step2, 0)
        plsc.subcore_barrier()

        r0 = pl.multiple_of(sid * 624, 8)
        pltpu.sync_copy(acc_sh.at[pl.ds(r0, 624)], out_hbm.at[cid, pl.ds(r0, 624)])

        @pl.when(sid == _NS - 1)
        def _tail():
            pltpu.sync_copy(acc_sh.at[pl.ds(9984, 16)], out_hbm.at[cid, pl.ds(9984, 16)])

    return k


def _degrees():
    D = 128

    @functools.partial(
        pl.kernel,
        out_type=jax.ShapeDtypeStruct((_NC, _N, D), jnp.float32),
        mesh=_sc_mesh(),
        scratch_types=[
            pltpu.VMEM((_NCH, _C), jnp.int32),
            pltpu.VMEM((_NCH, _C), jnp.int32),
            pltpu.VMEM((_C, D), jnp.float32),
            pltpu.VMEM((_C, D), jnp.float32),
            pltpu.VMEM_SHARED((_N, D), jnp.float32),
            pltpu.SemaphoreType.DMA,
        ],
    )
    def k(src_hbm, dst_hbm, out_hbm, src_v, dst_v, e0_v, e1_v, acc_sh, ss):
        cid = lax.axis_index("c")
        sid = lax.axis_index("s")
        wid = cid * _NS + sid

        _zero_rows(e0_v, _C, D)
        for b in range(_RPT // _C):
            pltpu.sync_copy(e0_v, acc_sh.at[pl.ds(sid * _RPT + b * _C, _C)])

        lanes = lax.iota(jnp.int32, 16)

        def fill(r, carry):
            e0_v[r, pl.ds(0, 16)] = jnp.where(lanes == 0, 1.0, 0.0)
            for j in range(1, D // 16):
                e1_v[r, pl.ds(j * 16, 16)] = jnp.zeros((16,), jnp.float32)
            e1_v[r, pl.ds(0, 16)] = jnp.where(lanes == 1, 1.0, 0.0)
            return carry

        lax.fori_loop(0, _C, fill, 0)
        plsc.subcore_barrier()

        pltpu.sync_copy(src_hbm.at[wid], src_v)
        pltpu.sync_copy(dst_hbm.at[wid], dst_v)

        def step(c, carry):
            pltpu.async_copy(e0_v, acc_sh.at[src_v.at[c]], ss, add=True)
            pltpu.sync_copy(e1_v, acc_sh.at[dst_v.at[c]], add=True)
            pltpu.make_async_copy(e0_v, acc_sh.at[src_v.at[c]], ss).wait()
            return carry

        lax.fori_loop(0, _NCH, step, 0)
        plsc.subcore_barrier()

        r0 = pl.multiple_of(sid * 624, 8)
        pltpu.sync_copy(acc_sh.at[pl.ds(r0, 624)], out_hbm.at[cid, pl.ds(r0, 624)])

        @pl.when(sid == _NS - 1)
        def _tail():
            pltpu.sync_copy(acc_sh.at[pl.ds(9984, 16)], out_hbm.at[cid, pl.ds(9984, 16)])

    return k


_BM = 1000


def _norm_body(s0, s1, d0, d1, ns_o, nd_o):
    ns_o[...] = lax.rsqrt(jnp.maximum(s0[...] + s1[...], 1.0))
    nd_o[...] = lax.rsqrt(jnp.maximum(d0[...] + d1[...], 1.0))


def _normalizers(s0, s1, d0, d1):
    return pl.pallas_call(
        _norm_body,
        out_shape=(
            jax.ShapeDtypeStruct((_N, 1), jnp.float32),
            jax.ShapeDtypeStruct((_N, 1), jnp.float32),
        ),
    )(s0, s1, d0, d1)


def _first_body(x, w, ns, o):
    y = lax.dot_general(x[...], w[...], (((1,), (0,)), ((), ())),
                        preferred_element_type=jnp.float32)
    o[...] = y * ns[...]


def _first_layer(x, w, ns):
    di, do = w.shape
    grid = _N // _BM
    return pl.pallas_call(
        _first_body,
        grid=(grid,),
        in_specs=[
            pl.BlockSpec((_BM, di), lambda i: (i, 0)),
            pl.BlockSpec((di, do), lambda i: (0, 0)),
            pl.BlockSpec((_BM, 1), lambda i: (i, 0)),
        ],
        out_specs=pl.BlockSpec((_BM, do), lambda i: (i, 0)),
        out_shape=jax.ShapeDtypeStruct((_N, do), jnp.float32),
    )(x, w, ns)


def _make_fused_body(pad):
    def _fused_body(p0, p1, nd, b, w, ns, o):
        h = (p0[...] + p1[...]) * nd[...] + b[...]
        h = jnp.maximum(h, 0.0)
        y = lax.dot_general(h, w[...], (((1,), (0,)), ((), ())),
                            preferred_element_type=jnp.float32)
        y = y * ns[...]
        if pad:
            y = jnp.concatenate([y, jnp.zeros((y.shape[0], pad), jnp.float32)], axis=1)
        o[...] = y
    return _fused_body


def _fused_layer(p0, p1, nd, b, w, ns, pad=0):
    di, do = w.shape
    grid = _N // _BM
    return pl.pallas_call(
        _make_fused_body(pad),
        grid=(grid,),
        in_specs=[
            pl.BlockSpec((_BM, di), lambda i: (i, 0)),
            pl.BlockSpec((_BM, di), lambda i: (i, 0)),
            pl.BlockSpec((_BM, 1), lambda i: (i, 0)),
            pl.BlockSpec((1, di), lambda i: (0, 0)),
            pl.BlockSpec((di, do), lambda i: (0, 0)),
            pl.BlockSpec((_BM, 1), lambda i: (i, 0)),
        ],
        out_specs=pl.BlockSpec((_BM, do + pad), lambda i: (i, 0)),
        out_shape=jax.ShapeDtypeStruct((_N, do + pad), jnp.float32),
    )(p0, p1, nd, b, w, ns)


def _zbranch_body(p0, p1, nd, b, wa, ws, ns, oz, oa, os_):
    zz = (p0[...] + p1[...]) * nd[...] + b[...]
    oz[...] = zz
    oa[...] = lax.dot_general(zz, wa[...], (((1,), (0,)), ((), ())),
                              preferred_element_type=jnp.float32) * ns[...]
    os_[...] = lax.dot_general(zz, ws[...], (((1,), (0,)), ((), ())),
                               preferred_element_type=jnp.float32) * ns[...]


def _zbranch(p0, p1, nd, b, wa, ws, ns):
    di, do = wa.shape
    grid = _N // _BM
    return pl.pallas_call(
        _zbranch_body,
        grid=(grid,),
        in_specs=[
            pl.BlockSpec((_BM, di), lambda i: (i, 0)),
            pl.BlockSpec((_BM, di), lambda i: (i, 0)),
            pl.BlockSpec((_BM, 1), lambda i: (i, 0)),
            pl.BlockSpec((1, di), lambda i: (0, 0)),
            pl.BlockSpec((di, do), lambda i: (0, 0)),
            pl.BlockSpec((di, do), lambda i: (0, 0)),
            pl.BlockSpec((_BM, 1), lambda i: (i, 0)),
        ],
        out_specs=[
            pl.BlockSpec((_BM, di), lambda i: (i, 0)),
            pl.BlockSpec((_BM, do), lambda i: (i, 0)),
            pl.BlockSpec((_BM, do), lambda i: (i, 0)),
        ],
        out_shape=[
            jax.ShapeDtypeStruct((_N, di), jnp.float32),
            jax.ShapeDtypeStruct((_N, do), jnp.float32),
            jax.ShapeDtypeStruct((_N, do), jnp.float32),
        ],
    )(p0, p1, nd, b, wa, ws, ns)


def _final_body(p0, p1, nd, b, o):
    o[...] = (p0[...] + p1[...]) * nd[...] + b[...]


def _final_layer(p0, p1, nd, b):
    d = p0.shape[1]
    grid = _N // _BM
    return pl.pallas_call(
        _final_body,
        grid=(grid,),
        in_specs=[
            pl.BlockSpec((_BM, d), lambda i: (i, 0)),
            pl.BlockSpec((_BM, d), lambda i: (i, 0)),
            pl.BlockSpec((_BM, 1), lambda i: (i, 0)),
            pl.BlockSpec((1, d), lambda i: (0, 0)),
        ],
        out_specs=pl.BlockSpec((_BM, d), lambda i: (i, 0)),
        out_shape=jax.ShapeDtypeStruct((_N, d), jnp.float32),
    )(p0, p1, nd, b)


_BA = 400


def _adj_body(zi, zjt, o):
    t = lax.dot_general(zi[...], zjt[...], (((1,), (0,)), ((), ())),
                        preferred_element_type=jnp.float32)
    o[...] = 1.0 / (1.0 + jnp.exp(-t))


def _adjacency(z):
    d = z.shape[1]
    g = _N // _BA
    return pl.pallas_call(
        _adj_body,
        grid=(g,),
        in_specs=[
            pl.BlockSpec((_BA, d), lambda i: (i, 0)),
            pl.BlockSpec((d, _N), lambda i: (0, 0)),
        ],
        out_specs=pl.BlockSpec((_BA, _N), lambda i: (i, 0)),
        out_shape=jax.ShapeDtypeStruct((_N, _N), jnp.float32),
    )(z, z.T)


def kernel(features, edge_index,
           enc_W0, enc_b0, enc_W1, enc_b1, enc_W2, enc_b2,
           dea_W0, dea_b0, dea_W1, dea_b1, dea_W2, dea_b2,
           des_W0, des_b0, des_W1, des_b1):
    src = edge_index[0].reshape(_NW, _NCH, _C)
    dst = edge_index[1].reshape(_NW, _NCH, _C)

    degp = _degrees()(src, dst)
    nsrc, ndst = _normalizers(degp[0, :, 0:1], degp[1, :, 0:1],
                              degp[0, :, 1:2], degp[1, :, 1:2])

    mp128 = _msgpass(128)

    y = _first_layer(features, enc_W0, nsrc)
    p = mp128(y, src, dst)
    y = _fused_layer(p[0], p[1], ndst, enc_b0.reshape(1, -1), enc_W1, nsrc)
    p = mp128(y, src, dst)
    y = _fused_layer(p[0], p[1], ndst, enc_b1.reshape(1, -1), enc_W2, nsrc, pad=64)
    p = mp128(y, src, dst)
    z, y_dea, y_des = _zbranch(p[0][:, :64], p[1][:, :64], ndst,
                               enc_b2.reshape(1, -1), dea_W0, des_W0, nsrc)

    p = mp128(y_des, src, dst)
    y = _fused_layer(p[0], p[1], ndst, des_b0.reshape(1, -1), des_W1, nsrc)
    p = mp128(y, src, dst)
    z_ = _final_layer(p[0], p[1], ndst, des_b1.reshape(1, -1))
    adj = _adjacency(z_)

    p = mp128(y_dea, src, dst)
    y = _fused_layer(p[0], p[1], ndst, dea_b0.reshape(1, -1), dea_W1, nsrc)
    p = mp128(y, src, dst)
    y = _fused_layer(p[0], p[1], ndst, dea_b1.reshape(1, -1), dea_W2, nsrc)
    p = mp128(y, src, dst)
    recon = _final_layer(p[0], p[1], ndst, dea_b2.reshape(1, -1))

    return z, recon, adj


# speedup vs baseline: 1.2020x; 1.0046x over previous
import functools

import jax
import jax.numpy as jnp
from jax import lax
from jax.experimental import pallas as pl
from jax.experimental.pallas import tpu as pltpu
from jax.experimental.pallas import tpu_sc as plsc

_N = 10000
_E = 160000
_NC = 2
_NS = 16
_NW = _NC * _NS
_EPW = _E // _NW
_C = 125
_NCH = _EPW // _C
_RPT = _N // _NS


def _sc_mesh():
    return plsc.VectorSubcoreMesh(
        core_axis_name="c", subcore_axis_name="s", num_cores=_NC, num_subcores=_NS
    )


def _zero_rows(ref, nrows, ncols):
    def body(r, carry):
        for j in range(ncols // 16):
            ref[r, pl.ds(j * 16, 16)] = jnp.zeros((16,), jnp.float32)
        return carry

    lax.fori_loop(0, nrows, body, 0)


def _msgpass(D):
    @functools.partial(
        pl.kernel,
        out_type=jax.ShapeDtypeStruct((_NC, _N, D), jnp.float32),
        mesh=_sc_mesh(),
        scratch_types=[
            pltpu.VMEM((_NCH, _C), jnp.int32),
            pltpu.VMEM((_NCH, _C), jnp.int32),
            pltpu.VMEM((_C, D), jnp.float32),
            pltpu.VMEM((_C, D), jnp.float32),
            pltpu.VMEM_SHARED((_N, D), jnp.float32),
            pltpu.SemaphoreType.DMA,
            pltpu.SemaphoreType.DMA,
            pltpu.SemaphoreType.DMA,
        ],
    )
    def k(y_hbm, src_hbm, dst_hbm, out_hbm, src_v, dst_v,
          rows0, rows1, acc_sh, gs0, gs1, zs):
        cid = lax.axis_index("c")
        sid = lax.axis_index("s")
        wid = cid * _NS + sid

        pltpu.async_copy(src_hbm.at[wid], src_v, gs0)
        pltpu.async_copy(dst_hbm.at[wid], dst_v, gs1)
        _zero_rows(rows0, _C, D)
        for b in range(_RPT // _C):
            pltpu.async_copy(rows0, acc_sh.at[pl.ds(sid * _RPT + b * _C, _C)], zs)
        for b in range(_RPT // _C):
            pltpu.make_async_copy(rows0, acc_sh.at[pl.ds(sid * _RPT + b * _C, _C)], zs).wait()
        pltpu.make_async_copy(src_hbm.at[wid], src_v, gs0).wait()
        pltpu.make_async_copy(dst_hbm.at[wid], dst_v, gs1).wait()
        plsc.subcore_barrier()

        pltpu.async_copy(y_hbm.at[src_v.at[0]], rows0, gs0)
        pltpu.async_copy(y_hbm.at[src_v.at[1]], rows1, gs1)

        def step2(i, carry):
            c = i * 2
            not_last = i < (_NCH // 2 - 1)
            pltpu.make_async_copy(y_hbm.at[src_v.at[c]], rows0, gs0).wait()
            pltpu.sync_copy(rows0, acc_sh.at[dst_v.at[c]], add=True)

            @pl.when(not_last)
            def _g0():
                pltpu.async_copy(y_hbm.at[src_v.at[c + 2]], rows0, gs0)

            pltpu.make_async_copy(y_hbm.at[src_v.at[c + 1]], rows1, gs1).wait()
            pltpu.sync_copy(rows1, acc_sh.at[dst_v.at[c + 1]], add=True)

            @pl.when(not_last)
            def _g1():
                pltpu.async_copy(y_hbm.at[src_v.at[c + 3]], rows1, gs1)

            return carry

        lax.fori_loop(0, _NCH // 2, step2, 0)
        plsc.subcore_barrier()

        r0 = pl.multiple_of(sid * 624, 8)
        pltpu.sync_copy(acc_sh.at[pl.ds(r0, 624)], out_hbm.at[cid, pl.ds(r0, 624)])

        @pl.when(sid == _NS - 1)
        def _tail():
            pltpu.sync_copy(acc_sh.at[pl.ds(9984, 16)], out_hbm.at[cid, pl.ds(9984, 16)])

    return k


def _degrees():
    D = 128

    @functools.partial(
        pl.kernel,
        out_type=jax.ShapeDtypeStruct((_NC, _N, D), jnp.float32),
        mesh=_sc_mesh(),
        scratch_types=[
            pltpu.VMEM((_NCH, _C), jnp.int32),
            pltpu.VMEM((_NCH, _C), jnp.int32),
            pltpu.VMEM((_C, D), jnp.float32),
            pltpu.VMEM((_C, D), jnp.float32),
            pltpu.VMEM_SHARED((_N, D), jnp.float32),
            pltpu.SemaphoreType.DMA,
        ],
    )
    def k(src_hbm, dst_hbm, out_hbm, src_v, dst_v, e0_v, e1_v, acc_sh, ss):
        cid = lax.axis_index("c")
        sid = lax.axis_index("s")
        wid = cid * _NS + sid

        pltpu.async_copy(src_hbm.at[wid], src_v, ss)
        _zero_rows(e0_v, _C, D)
        for b in range(_RPT // _C):
            pltpu.async_copy(e0_v, acc_sh.at[pl.ds(sid * _RPT + b * _C, _C)], ss)

        lanes = lax.iota(jnp.int32, 16)

        def fill(r, carry):
            for j in range(1, D // 16):
                e1_v[r, pl.ds(j * 16, 16)] = jnp.zeros((16,), jnp.float32)
            e1_v[r, pl.ds(0, 16)] = jnp.where(lanes == 1, 1.0, 0.0)
            return carry

        lax.fori_loop(0, _C, fill, 0)
        pltpu.sync_copy(dst_hbm.at[wid], dst_v)
        pltpu.make_async_copy(src_hbm.at[wid], src_v, ss).wait()
        for b in range(_RPT // _C):
            pltpu.make_async_copy(e0_v, acc_sh.at[pl.ds(sid * _RPT + b * _C, _C)], ss).wait()

        def fill0(r, carry):
            e0_v[r, pl.ds(0, 16)] = jnp.where(lanes == 0, 1.0, 0.0)
            return carry

        lax.fori_loop(0, _C, fill0, 0)
        plsc.subcore_barrier()

        def step(c, carry):
            pltpu.async_copy(e0_v, acc_sh.at[src_v.at[c]], ss, add=True)
            pltpu.sync_copy(e1_v, acc_sh.at[dst_v.at[c]], add=True)
            pltpu.make_async_copy(e0_v, acc_sh.at[src_v.at[c]], ss).wait()
            return carry

        lax.fori_loop(0, _NCH, step, 0)
        plsc.subcore_barrier()

        r0 = pl.multiple_of(sid * 624, 8)
        pltpu.sync_copy(acc_sh.at[pl.ds(r0, 624)], out_hbm.at[cid, pl.ds(r0, 624)])

        @pl.when(sid == _NS - 1)
        def _tail():
            pltpu.sync_copy(acc_sh.at[pl.ds(9984, 16)], out_hbm.at[cid, pl.ds(9984, 16)])

    return k


_BM = 1000


def _norm_body(s0, s1, d0, d1, ns_o, nd_o):
    ns_o[...] = lax.rsqrt(jnp.maximum(s0[...] + s1[...], 1.0))
    nd_o[...] = lax.rsqrt(jnp.maximum(d0[...] + d1[...], 1.0))


def _normalizers(s0, s1, d0, d1):
    return pl.pallas_call(
        _norm_body,
        out_shape=(
            jax.ShapeDtypeStruct((_N, 1), jnp.float32),
            jax.ShapeDtypeStruct((_N, 1), jnp.float32),
        ),
    )(s0, s1, d0, d1)


def _first_body(x, w, ns, o):
    y = lax.dot_general(x[...], w[...], (((1,), (0,)), ((), ())),
                        preferred_element_type=jnp.float32)
    o[...] = y * ns[...]


def _first_layer(x, w, ns):
    di, do = w.shape
    grid = _N // _BM
    return pl.pallas_call(
        _first_body,
        grid=(grid,),
        in_specs=[
            pl.BlockSpec((_BM, di), lambda i: (i, 0)),
            pl.BlockSpec((di, do), lambda i: (0, 0)),
            pl.BlockSpec((_BM, 1), lambda i: (i, 0)),
        ],
        out_specs=pl.BlockSpec((_BM, do), lambda i: (i, 0)),
        out_shape=jax.ShapeDtypeStruct((_N, do), jnp.float32),
    )(x, w, ns)


def _make_fused_body(pad):
    def _fused_body(p0, p1, nd, b, w, ns, o):
        h = (p0[...] + p1[...]) * nd[...] + b[...]
        h = jnp.maximum(h, 0.0)
        y = lax.dot_general(h, w[...], (((1,), (0,)), ((), ())),
                            preferred_element_type=jnp.float32)
        y = y * ns[...]
        if pad:
            y = jnp.concatenate([y, jnp.zeros((y.shape[0], pad), jnp.float32)], axis=1)
        o[...] = y
    return _fused_body


def _fused_layer(p0, p1, nd, b, w, ns, pad=0):
    di, do = w.shape
    grid = _N // _BM
    return pl.pallas_call(
        _make_fused_body(pad),
        grid=(grid,),
        in_specs=[
            pl.BlockSpec((_BM, di), lambda i: (i, 0)),
            pl.BlockSpec((_BM, di), lambda i: (i, 0)),
            pl.BlockSpec((_BM, 1), lambda i: (i, 0)),
            pl.BlockSpec((1, di), lambda i: (0, 0)),
            pl.BlockSpec((di, do), lambda i: (0, 0)),
            pl.BlockSpec((_BM, 1), lambda i: (i, 0)),
        ],
        out_specs=pl.BlockSpec((_BM, do + pad), lambda i: (i, 0)),
        out_shape=jax.ShapeDtypeStruct((_N, do + pad), jnp.float32),
    )(p0, p1, nd, b, w, ns)


def _zbranch_body(p0, p1, nd, b, wa, ws, ns, oz, oa, os_):
    zz = (p0[...] + p1[...]) * nd[...] + b[...]
    oz[...] = zz
    oa[...] = lax.dot_general(zz, wa[...], (((1,), (0,)), ((), ())),
                              preferred_element_type=jnp.float32) * ns[...]
    os_[...] = lax.dot_general(zz, ws[...], (((1,), (0,)), ((), ())),
                               preferred_element_type=jnp.float32) * ns[...]


def _zbranch(p0, p1, nd, b, wa, ws, ns):
    di, do = wa.shape
    grid = _N // _BM
    return pl.pallas_call(
        _zbranch_body,
        grid=(grid,),
        in_specs=[
            pl.BlockSpec((_BM, di), lambda i: (i, 0)),
            pl.BlockSpec((_BM, di), lambda i: (i, 0)),
            pl.BlockSpec((_BM, 1), lambda i: (i, 0)),
            pl.BlockSpec((1, di), lambda i: (0, 0)),
            pl.BlockSpec((di, do), lambda i: (0, 0)),
            pl.BlockSpec((di, do), lambda i: (0, 0)),
            pl.BlockSpec((_BM, 1), lambda i: (i, 0)),
        ],
        out_specs=[
            pl.BlockSpec((_BM, di), lambda i: (i, 0)),
            pl.BlockSpec((_BM, do), lambda i: (i, 0)),
            pl.BlockSpec((_BM, do), lambda i: (i, 0)),
        ],
        out_shape=[
            jax.ShapeDtypeStruct((_N, di), jnp.float32),
            jax.ShapeDtypeStruct((_N, do), jnp.float32),
            jax.ShapeDtypeStruct((_N, do), jnp.float32),
        ],
    )(p0, p1, nd, b, wa, ws, ns)


def _final_body(p0, p1, nd, b, o):
    o[...] = (p0[...] + p1[...]) * nd[...] + b[...]


def _final_layer(p0, p1, nd, b):
    d = p0.shape[1]
    grid = _N // _BM
    return pl.pallas_call(
        _final_body,
        grid=(grid,),
        in_specs=[
            pl.BlockSpec((_BM, d), lambda i: (i, 0)),
            pl.BlockSpec((_BM, d), lambda i: (i, 0)),
            pl.BlockSpec((_BM, 1), lambda i: (i, 0)),
            pl.BlockSpec((1, d), lambda i: (0, 0)),
        ],
        out_specs=pl.BlockSpec((_BM, d), lambda i: (i, 0)),
        out_shape=jax.ShapeDtypeStruct((_N, d), jnp.float32),
    )(p0, p1, nd, b)


_BA = 400


def _adj_body(zi, zjt, o):
    t = lax.dot_general(zi[...], zjt[...], (((1,), (0,)), ((), ())),
                        preferred_element_type=jnp.float32)
    o[...] = 1.0 / (1.0 + jnp.exp(-t))


def _adjacency(z):
    d = z.shape[1]
    g = _N // _BA
    return pl.pallas_call(
        _adj_body,
        grid=(g,),
        in_specs=[
            pl.BlockSpec((_BA, d), lambda i: (i, 0)),
            pl.BlockSpec((d, _N), lambda i: (0, 0)),
        ],
        out_specs=pl.BlockSpec((_BA, _N), lambda i: (i, 0)),
        out_shape=jax.ShapeDtypeStruct((_N, _N), jnp.float32),
    )(z, z.T)


def kernel(features, edge_index,
           enc_W0, enc_b0, enc_W1, enc_b1, enc_W2, enc_b2,
           dea_W0, dea_b0, dea_W1, dea_b1, dea_W2, dea_b2,
           des_W0, des_b0, des_W1, des_b1):
    src = edge_index[0].reshape(_NW, _NCH, _C)
    dst = edge_index[1].reshape(_NW, _NCH, _C)

    degp = _degrees()(src, dst)
    nsrc, ndst = _normalizers(degp[0, :, 0:1], degp[1, :, 0:1],
                              degp[0, :, 1:2], degp[1, :, 1:2])

    mp128 = _msgpass(128)

    y = _first_layer(features, enc_W0, nsrc)
    p = mp128(y, src, dst)
    y = _fused_layer(p[0], p[1], ndst, enc_b0.reshape(1, -1), enc_W1, nsrc)
    p = mp128(y, src, dst)
    y = _fused_layer(p[0], p[1], ndst, enc_b1.reshape(1, -1), enc_W2, nsrc, pad=64)
    p = mp128(y, src, dst)
    z, y_dea, y_des = _zbranch(p[0][:, :64], p[1][:, :64], ndst,
                               enc_b2.reshape(1, -1), dea_W0, des_W0, nsrc)

    p = mp128(y_des, src, dst)
    y = _fused_layer(p[0], p[1], ndst, des_b0.reshape(1, -1), des_W1, nsrc)
    p = mp128(y, src, dst)
    z_ = _final_layer(p[0], p[1], ndst, des_b1.reshape(1, -1))
    adj = _adjacency(z_)

    p = mp128(y_dea, src, dst)
    y = _fused_layer(p[0], p[1], ndst, dea_b0.reshape(1, -1), dea_W1, nsrc)
    p = mp128(y, src, dst)
    y = _fused_layer(p[0], p[1], ndst, dea_b1.reshape(1, -1), dea_W2, nsrc)
    p = mp128(y, src, dst)
    recon = _final_layer(p[0], p[1], ndst, dea_b2.reshape(1, -1))

    return z, recon, adj
